# Initial kernel scaffold; baseline (speedup 1.0000x reference)
import jax
import jax.numpy as jnp
from jax.experimental import pallas as pl


def kernel(uid_tensor, gender_tensor, age_tensor, job_tensor, movies_tensor, genres_tensor, title_tensor, uid_table, gender_table, age_table, job_table, movie_table, genre_table, title_table, W_uid, b_uid, W_gender, b_gender, W_age, b_age, W_job, b_job, W_user, b_user, W_mid, b_mid, W_genre, b_genre, W_title, b_title, W_movie, b_movie):
    raise NotImplementedError("write your pallas kernel here")


# speedup vs baseline: 3.8256x; 3.8256x over previous
import functools

import jax
import jax.numpy as jnp
from jax import lax
from jax.experimental import pallas as pl
from jax.experimental.pallas import tpu as pltpu
from jax.experimental.pallas import tpu_sc as plsc

B = 16384
D_ID = 64
D_CAT = 32
TITLE_LEN = 15
GENRE_LEN = 18
N_GENRES = 19

NC = 2
NS = 16
NW = NC * NS
BPW = B // NW


def _sc_gather(uid_idx, movie_idx, title_idx_t, uid_table, movie_table,
               title_table):
    mesh = plsc.VectorSubcoreMesh(core_axis_name="c", subcore_axis_name="s")
    out_type = (
        jax.ShapeDtypeStruct((B, D_ID), jnp.float32),
        jax.ShapeDtypeStruct((B, D_ID), jnp.float32),
        jax.ShapeDtypeStruct((B, D_CAT), jnp.float32),
    )
    scratch = dict(
        uidx_v=pltpu.VMEM((BPW,), jnp.int32),
        midx_v=pltpu.VMEM((BPW,), jnp.int32),
        tidx_v=[pltpu.VMEM((BPW,), jnp.int32) for _ in range(TITLE_LEN)],
        row_v=pltpu.VMEM((BPW, D_ID), jnp.float32),
        tacc_v=pltpu.VMEM((BPW, D_CAT), jnp.float32),
        tbuf_a=pltpu.VMEM((BPW, D_CAT), jnp.float32),
        tbuf_b=pltpu.VMEM((BPW, D_CAT), jnp.float32),
        sem_u=pltpu.SemaphoreType.DMA,
        sem_t0=pltpu.SemaphoreType.DMA,
        sem_a=pltpu.SemaphoreType.DMA,
        sem_b=pltpu.SemaphoreType.DMA,
    )

    @functools.partial(pl.kernel, mesh=mesh, out_type=out_type,
                       scratch_types=scratch,
                       compiler_params=pltpu.CompilerParams(
                           use_tc_tiling_on_sc=False))
    def k(uid_idx_hbm, movie_idx_hbm, title_idx_hbm,
          uid_tbl, movie_tbl, title_tbl,
          uid_out, movie_out, title_out,
          uidx_v, midx_v, tidx_v, row_v, tacc_v, tbuf_a, tbuf_b,
          sem_u, sem_t0, sem_a, sem_b):
        wid = lax.axis_index("s") * NC + lax.axis_index("c")
        base = wid * BPW

        pltpu.sync_copy(uid_idx_hbm.at[pl.ds(base, BPW)], uidx_v)
        pltpu.sync_copy(movie_idx_hbm.at[pl.ds(base, BPW)], midx_v)
        for j in range(TITLE_LEN):
            pltpu.sync_copy(title_idx_hbm.at[pl.ds(j * B + base, BPW)], tidx_v[j])

        def accum(buf):
            def body(i, carry):
                tacc_v[i, pl.ds(0, 16)] = tacc_v[i, pl.ds(0, 16)] + buf[i, pl.ds(0, 16)]
                tacc_v[i, pl.ds(16, 16)] = tacc_v[i, pl.ds(16, 16)] + buf[i, pl.ds(16, 16)]
                return carry
            lax.fori_loop(0, BPW, body, 0)

        bufs = (tbuf_a, tbuf_b)
        sems = (sem_a, sem_b)

        h_uid = pltpu.async_copy(uid_tbl.at[uidx_v], row_v, sem_u)
        h_t0 = pltpu.async_copy(title_tbl.at[tidx_v[0]], tacc_v, sem_t0)
        hs = {}
        hs[1] = pltpu.async_copy(title_tbl.at[tidx_v[1]], bufs[1 % 2], sems[1 % 2])
        hs[2] = pltpu.async_copy(title_tbl.at[tidx_v[2]], bufs[2 % 2], sems[2 % 2])
        h_t0.wait()
        for j in range(1, TITLE_LEN):
            hs[j].wait()
            accum(bufs[j % 2])
            if j == 1:
                h_uid.wait()
                pltpu.sync_copy(row_v, uid_out.at[pl.ds(base, BPW)])
                h_uid = pltpu.async_copy(movie_tbl.at[midx_v], row_v, sem_u)
            if j + 2 < TITLE_LEN:
                hs[j + 2] = pltpu.async_copy(
                    title_tbl.at[tidx_v[j + 2]], bufs[j % 2], sems[j % 2])
        pltpu.sync_copy(tacc_v, title_out.at[pl.ds(base, BPW)])
        h_uid.wait()
        pltpu.sync_copy(row_v, movie_out.at[pl.ds(base, BPW)])

    return k(uid_idx, movie_idx, title_idx_t, uid_table, movie_table,
             title_table)


BB = 2048


def _tc_body(uid_rows, mov_rows, tit_sum, gender, age, job, genres,
             gender_tbl, age_tbl, job_tbl, genre_tbl,
             W_uid, b_uid, W_gender, b_gender, W_age, b_age, W_job, b_job,
             W_user, b_user, W_mid, b_mid, W_genre, b_genre,
             W_title, b_title, W_movie, b_movie, out_ref):
    f32 = jnp.float32

    def dot(a, b):
        return jax.lax.dot(a, b, preferred_element_type=f32)

    def onehot(idx_col, n):
        iota = lax.broadcasted_iota(jnp.int32, (BB, n), 1)
        return (idx_col == iota).astype(f32)

    u_uid = jnp.maximum(dot(uid_rows[...], W_uid[...]) + b_uid[...], 0.0)
    u_gen = jnp.maximum(
        dot(dot(onehot(gender[...], 2), gender_tbl[...]), W_gender[...])
        + b_gender[...], 0.0)
    u_age = jnp.maximum(
        dot(dot(onehot(age[...], 7), age_tbl[...]), W_age[...])
        + b_age[...], 0.0)
    u_job = jnp.maximum(
        dot(dot(onehot(job[...], 21), job_tbl[...]), W_job[...])
        + b_job[...], 0.0)
    Wu = W_user[...]
    user = jnp.tanh(dot(u_uid, Wu[0:64]) + dot(u_gen, Wu[64:128])
                    + dot(u_age, Wu[128:192]) + dot(u_job, Wu[192:256])
                    + b_user[...])

    m_id = jnp.maximum(dot(mov_rows[...], W_mid[...]) + b_mid[...], 0.0)
    g = genres[...]
    iota_g = lax.broadcasted_iota(jnp.int32, (BB, N_GENRES), 1)
    counts = jnp.zeros((BB, N_GENRES), f32)
    for j in range(GENRE_LEN):
        counts = counts + (g[:, j:j + 1] == iota_g).astype(f32)
    genre_vec = dot(counts, genre_tbl[...])
    m_genre = jnp.maximum(dot(genre_vec, W_genre[...]) + b_genre[...], 0.0)
    m_title = jnp.maximum(
        dot(tit_sum[...] * (1.0 / TITLE_LEN), W_title[...]) + b_title[...], 0.0)
    Wm = W_movie[...]
    movie = jnp.tanh(dot(m_id, Wm[0:64]) + dot(m_genre, Wm[64:128])
                     + dot(m_title, Wm[128:192]) + b_movie[...])

    out_ref[...] = jnp.sum(user * movie, axis=1, keepdims=True)


def _tc_dense(uid_rows, mov_rows, tit_sum, gender, age, job, genres,
              gender_tbl, age_tbl, job_tbl, genre_tbl, *weights):
    grid = (B // BB,)

    def blk(shape_bb):
        return pl.BlockSpec(shape_bb, lambda i: (i, 0))

    def full(x):
        return pl.BlockSpec(x.shape, lambda i: (0,) * x.ndim)

    in_specs = [
        blk((BB, D_ID)), blk((BB, D_ID)), blk((BB, D_CAT)),
        blk((BB, 1)), blk((BB, 1)), blk((BB, 1)), blk((BB, GENRE_LEN)),
        full(gender_tbl), full(age_tbl), full(job_tbl), full(genre_tbl),
    ] + [full(w) for w in weights]

    return pl.pallas_call(
        _tc_body,
        grid=grid,
        in_specs=in_specs,
        out_specs=pl.BlockSpec((BB, 1), lambda i: (i, 0)),
        out_shape=jax.ShapeDtypeStruct((B, 1), jnp.float32),
    )(uid_rows, mov_rows, tit_sum, gender, age, job, genres,
      gender_tbl, age_tbl, job_tbl, genre_tbl, *weights)


def kernel(uid_tensor, gender_tensor, age_tensor, job_tensor, movies_tensor,
           genres_tensor, title_tensor, uid_table, gender_table, age_table,
           job_table, movie_table, genre_table, title_table,
           W_uid, b_uid, W_gender, b_gender, W_age, b_age, W_job, b_job,
           W_user, b_user, W_mid, b_mid, W_genre, b_genre,
           W_title, b_title, W_movie, b_movie):
    title_idx_t = jnp.transpose(title_tensor.astype(jnp.int32)).reshape(-1)
    uid_rows, mov_rows, tit_sum = _sc_gather(
        uid_tensor.astype(jnp.int32), movies_tensor.astype(jnp.int32),
        title_idx_t, uid_table, movie_table, title_table)
    out = _tc_dense(
        uid_rows, mov_rows, tit_sum,
        gender_tensor.astype(jnp.int32).reshape(B, 1),
        age_tensor.astype(jnp.int32).reshape(B, 1),
        job_tensor.astype(jnp.int32).reshape(B, 1),
        genres_tensor.astype(jnp.int32),
        gender_table, age_table, job_table, genre_table,
        W_uid, b_uid.reshape(1, -1), W_gender, b_gender.reshape(1, -1),
        W_age, b_age.reshape(1, -1), W_job, b_job.reshape(1, -1),
        W_user, b_user.reshape(1, -1), W_mid, b_mid.reshape(1, -1),
        W_genre, b_genre.reshape(1, -1), W_title, b_title.reshape(1, -1),
        W_movie, b_movie.reshape(1, -1))
    return out


# speedup vs baseline: 3.8888x; 1.0165x over previous
import functools

import jax
import jax.numpy as jnp
from jax import lax
from jax.experimental import pallas as pl
from jax.experimental.pallas import tpu as pltpu
from jax.experimental.pallas import tpu_sc as plsc

B = 16384
D_ID = 64
D_CAT = 32
TITLE_LEN = 15
GENRE_LEN = 18
N_GENRES = 19

NC = 2
NS = 16
NW = NC * NS
BPW = B // NW
HALF = BPW // 2


def _sc_gather_wide(uid_idx, movie_idx, uid_tbl2, movie_tbl2):
    mesh = plsc.VectorSubcoreMesh(core_axis_name="c", subcore_axis_name="s")
    out_type = (
        jax.ShapeDtypeStruct((B, 128), jnp.float32),
        jax.ShapeDtypeStruct((B, 128), jnp.float32),
    )
    scratch = dict(
        uidx_v=pltpu.VMEM((BPW,), jnp.int32),
        midx_v=pltpu.VMEM((BPW,), jnp.int32),
        ua_v=pltpu.VMEM((HALF,), jnp.int32),
        ub_v=pltpu.VMEM((HALF,), jnp.int32),
        ma_v=pltpu.VMEM((HALF,), jnp.int32),
        mb_v=pltpu.VMEM((HALF,), jnp.int32),
        buf_a=pltpu.VMEM((HALF, 128), jnp.float32),
        buf_b=pltpu.VMEM((HALF, 128), jnp.float32),
        sem_a=pltpu.SemaphoreType.DMA,
        sem_b=pltpu.SemaphoreType.DMA,
    )

    @functools.partial(pl.kernel, mesh=mesh, out_type=out_type,
                       scratch_types=scratch)
    def k(uid_idx_hbm, movie_idx_hbm, uid_tbl, movie_tbl,
          uid_out, movie_out,
          uidx_v, midx_v, ua_v, ub_v, ma_v, mb_v, buf_a, buf_b,
          sem_a, sem_b):
        wid = lax.axis_index("s") * NC + lax.axis_index("c")
        base = wid * BPW

        pltpu.sync_copy(uid_idx_hbm.at[pl.ds(base, BPW)], uidx_v)
        pltpu.sync_copy(movie_idx_hbm.at[pl.ds(base, BPW)], midx_v)

        def shift_body(i, carry):
            s = pl.ds(i * 16, 16)
            h = pl.ds((i % (HALF // 16)) * 16, 16)
            u = lax.shift_right_logical(uidx_v[s], 1)
            m = lax.shift_right_logical(midx_v[s], 1)

            @pl.when(i < HALF // 16)
            def _():
                ua_v[h] = u
                ma_v[h] = m

            @pl.when(i >= HALF // 16)
            def _():
                ub_v[h] = u
                mb_v[h] = m
            return carry
        lax.fori_loop(0, BPW // 16, shift_body, 0)

        h0 = pltpu.async_copy(uid_tbl.at[ua_v], buf_a, sem_a)
        h1 = pltpu.async_copy(uid_tbl.at[ub_v], buf_b, sem_b)
        h0.wait()
        pltpu.sync_copy(buf_a, uid_out.at[pl.ds(base, HALF)])
        h2 = pltpu.async_copy(movie_tbl.at[ma_v], buf_a, sem_a)
        h1.wait()
        pltpu.sync_copy(buf_b, uid_out.at[pl.ds(base + HALF, HALF)])
        h3 = pltpu.async_copy(movie_tbl.at[mb_v], buf_b, sem_b)
        h2.wait()
        pltpu.sync_copy(buf_a, movie_out.at[pl.ds(base, HALF)])
        h3.wait()
        pltpu.sync_copy(buf_b, movie_out.at[pl.ds(base + HALF, HALF)])

    return k(uid_idx, movie_idx, uid_tbl2, movie_tbl2)


def _sc_title(title_idx_flat, title_table):
    mesh = plsc.VectorSubcoreMesh(core_axis_name="c", subcore_axis_name="s")
    out_type = jax.ShapeDtypeStruct((B, D_CAT), jnp.float32)
    scratch = dict(
        tidx_v=[pltpu.VMEM((BPW,), jnp.int32) for _ in range(TITLE_LEN)],
        tacc_v=pltpu.VMEM((BPW, D_CAT), jnp.float32),
        tbuf_a=pltpu.VMEM((BPW, D_CAT), jnp.float32),
        tbuf_b=pltpu.VMEM((BPW, D_CAT), jnp.float32),
        sem_t0=pltpu.SemaphoreType.DMA,
        sem_a=pltpu.SemaphoreType.DMA,
        sem_b=pltpu.SemaphoreType.DMA,
    )

    @functools.partial(pl.kernel, mesh=mesh, out_type=out_type,
                       scratch_types=scratch,
                       compiler_params=pltpu.CompilerParams(
                           use_tc_tiling_on_sc=False))
    def k(title_idx_hbm, title_tbl, title_out,
          tidx_v, tacc_v, tbuf_a, tbuf_b, sem_t0, sem_a, sem_b):
        wid = lax.axis_index("s") * NC + lax.axis_index("c")
        base = wid * BPW

        for j in range(TITLE_LEN):
            pltpu.sync_copy(title_idx_hbm.at[pl.ds(j * B + base, BPW)],
                            tidx_v[j])

        def accum(buf):
            def body(i, carry):
                tacc_v[i, pl.ds(0, 16)] = tacc_v[i, pl.ds(0, 16)] + buf[i, pl.ds(0, 16)]
                tacc_v[i, pl.ds(16, 16)] = tacc_v[i, pl.ds(16, 16)] + buf[i, pl.ds(16, 16)]
                return carry
            lax.fori_loop(0, BPW, body, 0)

        bufs = (tbuf_a, tbuf_b)
        sems = (sem_a, sem_b)
        h_t0 = pltpu.async_copy(title_tbl.at[tidx_v[0]], tacc_v, sem_t0)
        hs = {}
        hs[1] = pltpu.async_copy(title_tbl.at[tidx_v[1]], bufs[1], sems[1])
        hs[2] = pltpu.async_copy(title_tbl.at[tidx_v[2]], bufs[0], sems[0])
        h_t0.wait()
        for j in range(1, TITLE_LEN):
            hs[j].wait()
            accum(bufs[j % 2])
            if j + 2 < TITLE_LEN:
                hs[j + 2] = pltpu.async_copy(
                    title_tbl.at[tidx_v[j + 2]], bufs[j % 2], sems[j % 2])
        pltpu.sync_copy(tacc_v, title_out.at[pl.ds(base, BPW)])

    return k(title_idx_flat, title_table)


BB = 2048


def _tc_body(uid_rows, mov_rows, tit_sum, uid_col, mov_col,
             gender, age, job, genres,
             gender_tbl, age_tbl, job_tbl, genre_tbl,
             W_uid, b_uid, W_gender, b_gender, W_age, b_age, W_job, b_job,
             W_user, b_user, W_mid, b_mid, W_genre, b_genre,
             W_title, b_title, W_movie, b_movie, out_ref):
    f32 = jnp.float32

    def dot(a, b):
        return jax.lax.dot(a, b, preferred_element_type=f32)

    def onehot(idx_col, n):
        iota = lax.broadcasted_iota(jnp.int32, (BB, n), 1)
        return (idx_col == iota).astype(f32)

    def half_select(rows128, idx_col):
        odd = (idx_col & 1) == 1
        return jnp.where(odd, rows128[:, D_ID:2 * D_ID], rows128[:, 0:D_ID])

    uid_emb = half_select(uid_rows[...], uid_col[...])
    mov_emb = half_select(mov_rows[...], mov_col[...])

    u_uid = jnp.maximum(dot(uid_emb, W_uid[...]) + b_uid[...], 0.0)
    u_gen = jnp.maximum(
        dot(dot(onehot(gender[...], 2), gender_tbl[...]), W_gender[...])
        + b_gender[...], 0.0)
    u_age = jnp.maximum(
        dot(dot(onehot(age[...], 7), age_tbl[...]), W_age[...])
        + b_age[...], 0.0)
    u_job = jnp.maximum(
        dot(dot(onehot(job[...], 21), job_tbl[...]), W_job[...])
        + b_job[...], 0.0)
    Wu = W_user[...]
    user = jnp.tanh(dot(u_uid, Wu[0:64]) + dot(u_gen, Wu[64:128])
                    + dot(u_age, Wu[128:192]) + dot(u_job, Wu[192:256])
                    + b_user[...])

    m_id = jnp.maximum(dot(mov_emb, W_mid[...]) + b_mid[...], 0.0)
    g = genres[...]
    iota_g = lax.broadcasted_iota(jnp.int32, (BB, N_GENRES), 1)
    counts = jnp.zeros((BB, N_GENRES), f32)
    for j in range(GENRE_LEN):
        counts = counts + (g[:, j:j + 1] == iota_g).astype(f32)
    genre_vec = dot(counts, genre_tbl[...])
    m_genre = jnp.maximum(dot(genre_vec, W_genre[...]) + b_genre[...], 0.0)
    m_title = jnp.maximum(
        dot(tit_sum[...] * (1.0 / TITLE_LEN), W_title[...]) + b_title[...], 0.0)
    Wm = W_movie[...]
    movie = jnp.tanh(dot(m_id, Wm[0:64]) + dot(m_genre, Wm[64:128])
                     + dot(m_title, Wm[128:192]) + b_movie[...])

    out_ref[...] = jnp.sum(user * movie, axis=1, keepdims=True)


def _tc_dense(uid_rows, mov_rows, tit_sum, uid_col, mov_col,
              gender, age, job, genres,
              gender_tbl, age_tbl, job_tbl, genre_tbl, *weights):
    grid = (B // BB,)

    def blk(shape_bb):
        return pl.BlockSpec(shape_bb, lambda i: (i, 0))

    def full(x):
        return pl.BlockSpec(x.shape, lambda i: (0,) * x.ndim)

    in_specs = [
        blk((BB, 128)), blk((BB, 128)), blk((BB, D_CAT)),
        blk((BB, 1)), blk((BB, 1)),
        blk((BB, 1)), blk((BB, 1)), blk((BB, 1)), blk((BB, GENRE_LEN)),
        full(gender_tbl), full(age_tbl), full(job_tbl), full(genre_tbl),
    ] + [full(w) for w in weights]

    return pl.pallas_call(
        _tc_body,
        grid=grid,
        in_specs=in_specs,
        out_specs=pl.BlockSpec((BB, 1), lambda i: (i, 0)),
        out_shape=jax.ShapeDtypeStruct((B, 1), jnp.float32),
    )(uid_rows, mov_rows, tit_sum, uid_col, mov_col,
      gender, age, job, genres,
      gender_tbl, age_tbl, job_tbl, genre_tbl, *weights)


def kernel(uid_tensor, gender_tensor, age_tensor, job_tensor, movies_tensor,
           genres_tensor, title_tensor, uid_table, gender_table, age_table,
           job_table, movie_table, genre_table, title_table,
           W_uid, b_uid, W_gender, b_gender, W_age, b_age, W_job, b_job,
           W_user, b_user, W_mid, b_mid, W_genre, b_genre,
           W_title, b_title, W_movie, b_movie):
    uid_idx = uid_tensor.astype(jnp.int32)
    mov_idx = movies_tensor.astype(jnp.int32)
    title_idx_flat = jnp.transpose(title_tensor.astype(jnp.int32)).reshape(-1)

    uid_tbl2 = uid_table.reshape(-1, 128)
    mov_tbl2 = movie_table.reshape(-1, 128)

    uid_rows, mov_rows = _sc_gather_wide(uid_idx, mov_idx, uid_tbl2, mov_tbl2)
    tit_sum = _sc_title(title_idx_flat, title_table)

    out = _tc_dense(
        uid_rows, mov_rows, tit_sum,
        uid_idx.reshape(B, 1), mov_idx.reshape(B, 1),
        gender_tensor.astype(jnp.int32).reshape(B, 1),
        age_tensor.astype(jnp.int32).reshape(B, 1),
        job_tensor.astype(jnp.int32).reshape(B, 1),
        genres_tensor.astype(jnp.int32),
        gender_table, age_table, job_table, genre_table,
        W_uid, b_uid.reshape(1, -1), W_gender, b_gender.reshape(1, -1),
        W_age, b_age.reshape(1, -1), W_job, b_job.reshape(1, -1),
        W_user, b_user.reshape(1, -1), W_mid, b_mid.reshape(1, -1),
        W_genre, b_genre.reshape(1, -1), W_title, b_title.reshape(1, -1),
        W_movie, b_movie.reshape(1, -1))
    return out
